# baseline (speedup 1.0000x reference)
import functools

import jax
import jax.numpy as jnp
from jax import lax
from jax.experimental import pallas as pl
from jax.experimental.pallas import tpu as pltpu
from jax.experimental.pallas import tpu_sc as plsc

N = 10000
D = 128
E = 320000

NC = 2
NS = 16
L = 16

NPAD = 12288
HALF = NPAD // NC
NT = HALF // NS
NTV = NT // L
ECHUNK = E // NS
EV = ECHUNK // L

_PART_S = (N - HALF * (NC - 1)) // NT
_PART_ROWS = N - HALF * (NC - 1) - _PART_S * NT

_MESH = plsc.VectorSubcoreMesh(core_axis_name="c", subcore_axis_name="s")


def _rsqrt_f32(d):
    xi = lax.bitcast_convert_type(d, jnp.int32)
    yi = jnp.int32(0x5F3759DF) - (xi >> 1)
    y = lax.bitcast_convert_type(yi, jnp.float32)
    for _ in range(3):
        y = y * (1.5 - 0.5 * d * y * y)
    return y


@functools.partial(
    pl.kernel,
    out_type=(
        jax.ShapeDtypeStruct((NPAD,), jnp.float32),
        jax.ShapeDtypeStruct((NPAD,), jnp.float32),
    ),
    mesh=_MESH,
    scratch_types=[
        pltpu.VMEM((ECHUNK,), jnp.int32),
        pltpu.VMEM((HALF,), jnp.float32),
        pltpu.VMEM((NS * NT,), jnp.float32),
        pltpu.VMEM((NT, D), jnp.float32),
        pltpu.VMEM((NT * L,), jnp.float32),
        pltpu.VMEM((D,), jnp.float32),
        pltpu.VMEM((NT,), jnp.float32),
        pltpu.VMEM((NT,), jnp.float32),
        pltpu.VMEM((NT,), jnp.float32),
        pltpu.VMEM_SHARED((NS * HALF,), jnp.float32),
        pltpu.SemaphoreType.DMA,
        pltpu.SemaphoreType.DMA,
        pltpu.SemaphoreType.DMA,
    ],
    compiler_params=pltpu.CompilerParams(needs_layout_passes=False),
)
def _sc_degree(x_hbm, w_hbm, ei_hbm, g_out, dis_out,
               colv, hist, red, xsl, htmp, wsl, hsl, gsl, dsl, shared,
               semx, seme, semc):
    c = lax.axis_index("c")
    s = lax.axis_index("s")
    base = c * HALF
    row0 = base + s * NT

    full = row0 + NT <= N
    part = jnp.logical_not(full) & (row0 < N)

    @pl.when(full)
    def _():
        pltpu.async_copy(x_hbm.at[pl.ds(row0, NT), :], xsl, semx)

    @pl.when(part)
    def _():
        pltpu.async_copy(x_hbm.at[pl.ds(row0, _PART_ROWS), :],
                         xsl.at[pl.ds(0, _PART_ROWS), :], semx)

    pltpu.async_copy(ei_hbm.at[pl.ds(E + s * ECHUNK, ECHUNK)], colv, seme)
    pltpu.sync_copy(w_hbm, wsl)

    zero16 = jnp.zeros((L,), jnp.float32)

    @plsc.parallel_loop(0, HALF // L, unroll=8)
    def _(i):
        hist[pl.ds(i * L, L)] = zero16

    pltpu.make_async_copy(ei_hbm.at[pl.ds(E + s * ECHUNK, ECHUNK)],
                          colv, seme).wait()
    ones = jnp.ones((L,), jnp.float32)

    @plsc.parallel_loop(0, EV, unroll=4)
    def _(i):
        cols = colv[pl.ds(i * L, L)]
        loc = cols - base
        m = lax.bitcast_convert_type(loc, jnp.uint32) < jnp.uint32(HALF)
        plsc.addupdate_scatter(hist, [loc], ones, mask=m)

    pltpu.async_copy(hist, shared.at[pl.ds(s * HALF, HALF)], semc)

    @pl.when(full)
    def _():
        pltpu.make_async_copy(x_hbm.at[pl.ds(row0, NT), :], xsl, semx).wait()

    @pl.when(part)
    def _():
        pltpu.make_async_copy(x_hbm.at[pl.ds(row0, _PART_ROWS), :],
                              xsl.at[pl.ds(0, _PART_ROWS), :], semx).wait()

    lanes = lax.iota(jnp.int32, L)
    wvecs = [wsl[pl.ds(k * L, L)] for k in range(D // L)]

    @plsc.parallel_loop(0, NT, unroll=2)
    def _(r):
        a0 = xsl[r, pl.ds(0, L)] * wvecs[0]
        a1 = xsl[r, pl.ds(L, L)] * wvecs[1]
        for k in range(2, D // L, 2):
            a0 = a0 + xsl[r, pl.ds(k * L, L)] * wvecs[k]
            a1 = a1 + xsl[r, pl.ds((k + 1) * L, L)] * wvecs[k + 1]
        htmp[pl.ds(r * L, L)] = plsc.cumsum(a0 + a1)

    @plsc.parallel_loop(0, NTV)
    def _(j):
        idx = (lanes + j * L) * L + (L - 1)
        hsl[pl.ds(j * L, L)] = plsc.load_gather(htmp, [idx])

    pltpu.make_async_copy(hist, shared.at[pl.ds(s * HALF, HALF)], semc).wait()
    plsc.subcore_barrier()
    for t in range(NS):
        pltpu.async_copy(shared.at[pl.ds(t * HALF + s * NT, NT)],
                         red.at[pl.ds(t * NT, NT)], semc)
    for t in range(NS):
        pltpu.make_async_copy(shared.at[pl.ds(t * HALF + s * NT, NT)],
                              red.at[pl.ds(t * NT, NT)], semc).wait()

    @plsc.parallel_loop(0, NTV)
    def _(j):
        acc = red[pl.ds(j * L, L)]
        for t in range(1, NS):
            acc = acc + red[pl.ds(t * NT + j * L, L)]
        d = acc + 1.0
        y = _rsqrt_f32(d)
        dsl[pl.ds(j * L, L)] = y
        gsl[pl.ds(j * L, L)] = y * hsl[pl.ds(j * L, L)]

    pltpu.sync_copy(gsl, g_out.at[pl.ds(row0, NT)])
    pltpu.sync_copy(dsl, dis_out.at[pl.ds(row0, NT)])


@functools.partial(
    pl.kernel,
    out_type=jax.ShapeDtypeStruct((NPAD,), jnp.float32),
    mesh=_MESH,
    scratch_types=[
        pltpu.VMEM((ECHUNK,), jnp.int32),
        pltpu.VMEM((ECHUNK,), jnp.int32),
        pltpu.VMEM((NPAD,), jnp.float32),
        pltpu.VMEM((HALF,), jnp.float32),
        pltpu.VMEM((NS * NT,), jnp.float32),
        pltpu.VMEM((NT,), jnp.float32),
        pltpu.VMEM((NT,), jnp.float32),
        pltpu.VMEM((L,), jnp.float32),
        pltpu.VMEM_SHARED((NS * HALF,), jnp.float32),
        pltpu.SemaphoreType.DMA,
    ],
    compiler_params=pltpu.CompilerParams(needs_layout_passes=False),
)
def _sc_edges(ei_hbm, g_hbm, dis_hbm, b_hbm, out_hbm,
              rowv, colv, gv, spart, red, dsl, osl, bv, shared, sem):
    c = lax.axis_index("c")
    s = lax.axis_index("s")
    base = c * HALF

    pltpu.async_copy(g_hbm, gv, sem)
    pltpu.async_copy(ei_hbm.at[pl.ds(s * ECHUNK, ECHUNK)], rowv, sem)
    pltpu.async_copy(ei_hbm.at[pl.ds(E + s * ECHUNK, ECHUNK)], colv, sem)

    zero16 = jnp.zeros((L,), jnp.float32)

    @plsc.parallel_loop(0, HALF // L, unroll=8)
    def _(i):
        spart[pl.ds(i * L, L)] = zero16

    pltpu.make_async_copy(g_hbm, gv, sem).wait()
    pltpu.make_async_copy(ei_hbm.at[pl.ds(s * ECHUNK, ECHUNK)], rowv, sem).wait()
    pltpu.make_async_copy(ei_hbm.at[pl.ds(E + s * ECHUNK, ECHUNK)], colv, sem).wait()

    @plsc.parallel_loop(0, EV, unroll=4)
    def _(i):
        rows = rowv[pl.ds(i * L, L)]
        cols = colv[pl.ds(i * L, L)]
        gvals = plsc.load_gather(gv, [rows])
        loc = cols - base
        m = lax.bitcast_convert_type(loc, jnp.uint32) < jnp.uint32(HALF)
        plsc.addupdate_scatter(spart, [loc], gvals, mask=m)

    pltpu.sync_copy(spart, shared.at[pl.ds(s * HALF, HALF)])
    plsc.subcore_barrier()
    for t in range(NS):
        pltpu.sync_copy(shared.at[pl.ds(t * HALF + s * NT, NT)],
                        red.at[pl.ds(t * NT, NT)])
    pltpu.sync_copy(dis_hbm.at[pl.ds(base + s * NT, NT)], dsl)
    pltpu.sync_copy(b_hbm, bv)
    bval = bv[pl.ds(0, L)]

    @plsc.parallel_loop(0, NTV)
    def _(j):
        acc = red[pl.ds(j * L, L)]
        for t in range(1, NS):
            acc = acc + red[pl.ds(t * NT + j * L, L)]
        y = dsl[pl.ds(j * L, L)]
        gg = gv[pl.ds(base + s * NT + j * L, L)]
        o = y * acc + y * gg + bval
        osl[pl.ds(j * L, L)] = jnp.maximum(o, 0.0)

    pltpu.sync_copy(osl, out_hbm.at[pl.ds(base + s * NT, NT)])


@jax.jit
def kernel(x, edge_index, W, b):
    edge_index = edge_index.astype(jnp.int32).reshape(2 * E)
    w1 = W.astype(jnp.float32).reshape(D)
    b16 = jnp.broadcast_to(b.astype(jnp.float32).reshape(1), (L,))

    g, dis = _sc_degree(x, w1, edge_index)
    out_pad = _sc_edges(edge_index, g, dis, b16)
    return out_pad[:N].reshape(N, 1)


# speedup vs baseline: 1.1312x; 1.1312x over previous
import functools

import jax
import jax.numpy as jnp
from jax import lax
from jax.experimental import pallas as pl
from jax.experimental.pallas import tpu as pltpu
from jax.experimental.pallas import tpu_sc as plsc

N = 10000
D = 128
E = 320000

NC = 2
NS = 16
L = 16

NPAD = 12288
HALF = NPAD // NC
NT = HALF // NS
NTV = NT // L
ECHUNK = E // NS
EV = ECHUNK // L

_PART_ROWS = (N - HALF) % NT

_MESH = plsc.VectorSubcoreMesh(core_axis_name="c", subcore_axis_name="s")


def _rsqrt_f32(d):
    xi = lax.bitcast_convert_type(d, jnp.int32)
    yi = jnp.int32(0x5F3759DF) - (xi >> 1)
    y = lax.bitcast_convert_type(yi, jnp.float32)
    for _ in range(3):
        y = y * (1.5 - 0.5 * d * y * y)
    return y


@functools.partial(
    pl.kernel,
    out_type=(
        jax.ShapeDtypeStruct((NPAD,), jnp.float32),
        jax.ShapeDtypeStruct((NPAD,), jnp.float32),
    ),
    mesh=_MESH,
    scratch_types=[
        pltpu.VMEM((ECHUNK,), jnp.int32),
        pltpu.VMEM((ECHUNK,), jnp.int32),
        pltpu.VMEM((HALF,), jnp.float32),
        pltpu.VMEM((NS * NT,), jnp.float32),
        pltpu.VMEM((NT, D), jnp.float32),
        pltpu.VMEM((NPAD,), jnp.float32),
        pltpu.VMEM((D,), jnp.float32),
        pltpu.VMEM((NT,), jnp.float32),
        pltpu.VMEM((NT,), jnp.float32),
        pltpu.VMEM((NT,), jnp.float32),
        pltpu.VMEM((NT,), jnp.float32),
        pltpu.VMEM((L,), jnp.float32),
        pltpu.VMEM_SHARED((NS * HALF,), jnp.float32),
        pltpu.SemaphoreType.DMA,
        pltpu.SemaphoreType.DMA,
        pltpu.SemaphoreType.DMA,
        pltpu.SemaphoreType.REGULAR,
    ],
    compiler_params=pltpu.CompilerParams(needs_layout_passes=False),
)
def _sc_gcn(x_hbm, w_hbm, ei_hbm, b_hbm, out_hbm, g_hbm,
            rowv, colv, hist, red, xsl, gv, wsl, hsl, gsl, dsl, osl, bv,
            shared, semx, seme, semc, semg):
    c = lax.axis_index("c")
    s = lax.axis_index("s")
    base = c * HALF
    row0 = base + s * NT

    full = row0 + NT <= N
    part = jnp.logical_not(full) & (row0 < N)

    @pl.when(full)
    def _():
        pltpu.async_copy(x_hbm.at[pl.ds(row0, NT), :], xsl, semx)

    @pl.when(part)
    def _():
        pltpu.async_copy(x_hbm.at[pl.ds(row0, _PART_ROWS), :],
                         xsl.at[pl.ds(0, _PART_ROWS), :], semx)

    pltpu.async_copy(ei_hbm.at[pl.ds(E + s * ECHUNK, ECHUNK)], colv, seme)
    pltpu.async_copy(ei_hbm.at[pl.ds(s * ECHUNK, ECHUNK)], rowv, seme)
    pltpu.sync_copy(w_hbm, wsl)
    pltpu.sync_copy(b_hbm, bv)

    zero16 = jnp.zeros((L,), jnp.float32)

    @plsc.parallel_loop(0, HALF // L, unroll=8)
    def _(i):
        hist[pl.ds(i * L, L)] = zero16

    pltpu.make_async_copy(ei_hbm.at[pl.ds(E + s * ECHUNK, ECHUNK)],
                          colv, seme).wait()
    ones = jnp.ones((L,), jnp.float32)

    @plsc.parallel_loop(0, EV, unroll=4)
    def _(i):
        cols = colv[pl.ds(i * L, L)]
        loc = cols - base
        m = lax.bitcast_convert_type(loc, jnp.uint32) < jnp.uint32(HALF)
        plsc.addupdate_scatter(hist, [loc], ones, mask=m)

    pltpu.async_copy(hist, shared.at[pl.ds(s * HALF, HALF)], semc)

    @pl.when(full)
    def _():
        pltpu.make_async_copy(x_hbm.at[pl.ds(row0, NT), :], xsl, semx).wait()

    @pl.when(part)
    def _():
        pltpu.make_async_copy(x_hbm.at[pl.ds(row0, _PART_ROWS), :],
                              xsl.at[pl.ds(0, _PART_ROWS), :], semx).wait()

    lanes = lax.iota(jnp.int32, L)
    wvecs = [wsl[pl.ds(k * L, L)] for k in range(D // L)]

    @plsc.parallel_loop(0, NT, unroll=2)
    def _(r):
        a0 = xsl[r, pl.ds(0, L)] * wvecs[0]
        a1 = xsl[r, pl.ds(L, L)] * wvecs[1]
        for k in range(2, D // L, 2):
            a0 = a0 + xsl[r, pl.ds(k * L, L)] * wvecs[k]
            a1 = a1 + xsl[r, pl.ds((k + 1) * L, L)] * wvecs[k + 1]
        red[pl.ds(r * L, L)] = plsc.cumsum(a0 + a1)

    @plsc.parallel_loop(0, NTV)
    def _(j):
        idx = (lanes + j * L) * L + (L - 1)
        hsl[pl.ds(j * L, L)] = plsc.load_gather(red, [idx])

    pltpu.make_async_copy(hist, shared.at[pl.ds(s * HALF, HALF)], semc).wait()
    plsc.subcore_barrier()
    for t in range(NS):
        pltpu.async_copy(shared.at[pl.ds(t * HALF + s * NT, NT)],
                         red.at[pl.ds(t * NT, NT)], semc)
    for t in range(NS):
        pltpu.make_async_copy(shared.at[pl.ds(t * HALF + s * NT, NT)],
                              red.at[pl.ds(t * NT, NT)], semc).wait()

    @plsc.parallel_loop(0, NTV)
    def _(j):
        acc = red[pl.ds(j * L, L)]
        for t in range(1, NS):
            acc = acc + red[pl.ds(t * NT + j * L, L)]
        d = acc + 1.0
        y = _rsqrt_f32(d)
        dsl[pl.ds(j * L, L)] = y
        gsl[pl.ds(j * L, L)] = y * hsl[pl.ds(j * L, L)]

    pltpu.sync_copy(gsl, g_hbm.at[pl.ds(row0, NT)])

    plsc.subcore_barrier()

    @pl.when(s == 0)
    def _():
        pltpu.semaphore_signal(semg, 1, core_index=1 - c)
        pl.semaphore_wait(semg, 1)

    plsc.subcore_barrier()

    pltpu.async_copy(g_hbm, gv, semx)

    @plsc.parallel_loop(0, HALF // L, unroll=8)
    def _(i):
        hist[pl.ds(i * L, L)] = zero16

    pltpu.make_async_copy(ei_hbm.at[pl.ds(s * ECHUNK, ECHUNK)],
                          rowv, seme).wait()
    pltpu.make_async_copy(g_hbm, gv, semx).wait()

    @plsc.parallel_loop(0, EV, unroll=4)
    def _(i):
        rows = rowv[pl.ds(i * L, L)]
        cols = colv[pl.ds(i * L, L)]
        gvals = plsc.load_gather(gv, [rows])
        loc = cols - base
        m = lax.bitcast_convert_type(loc, jnp.uint32) < jnp.uint32(HALF)
        plsc.addupdate_scatter(hist, [loc], gvals, mask=m)

    pltpu.sync_copy(hist, shared.at[pl.ds(s * HALF, HALF)])
    plsc.subcore_barrier()
    for t in range(NS):
        pltpu.async_copy(shared.at[pl.ds(t * HALF + s * NT, NT)],
                         red.at[pl.ds(t * NT, NT)], semc)
    for t in range(NS):
        pltpu.make_async_copy(shared.at[pl.ds(t * HALF + s * NT, NT)],
                              red.at[pl.ds(t * NT, NT)], semc).wait()

    bval = bv[pl.ds(0, L)]

    @plsc.parallel_loop(0, NTV)
    def _(j):
        acc = red[pl.ds(j * L, L)]
        for t in range(1, NS):
            acc = acc + red[pl.ds(t * NT + j * L, L)]
        y = dsl[pl.ds(j * L, L)]
        o = y * acc + y * gsl[pl.ds(j * L, L)] + bval
        osl[pl.ds(j * L, L)] = jnp.maximum(o, 0.0)

    pltpu.sync_copy(osl, out_hbm.at[pl.ds(row0, NT)])


@jax.jit
def kernel(x, edge_index, W, b):
    ei = edge_index.astype(jnp.int32).reshape(2 * E)
    w1 = W.astype(jnp.float32).reshape(D)
    b16 = jnp.broadcast_to(b.astype(jnp.float32).reshape(1), (L,))

    out_pad, _ = _sc_gcn(x, w1, ei, b16)
    return out_pad[:N].reshape(N, 1)


# speedup vs baseline: 1.2139x; 1.0731x over previous
import functools

import jax
import jax.numpy as jnp
from jax import lax
from jax.experimental import pallas as pl
from jax.experimental.pallas import tpu as pltpu
from jax.experimental.pallas import tpu_sc as plsc

N = 10000
D = 128
E = 320000

NC = 2
NS = 16
L = 16

NPAD = 12288
HALF = NPAD // NC
NT = HALF // NS
NTV = NT // L
ECHUNK = E // NS
EV = ECHUNK // L
CHP = 20096

_PART_ROWS = (N - HALF) % NT

_MESH = plsc.VectorSubcoreMesh(core_axis_name="c", subcore_axis_name="s")


def _rsqrt_f32(d):
    xi = lax.bitcast_convert_type(d, jnp.int32)
    yi = jnp.int32(0x5F3759DF) - (xi >> 1)
    y = lax.bitcast_convert_type(yi, jnp.float32)
    for _ in range(3):
        y = y * (1.5 - 0.5 * d * y * y)
    return y


@functools.partial(
    pl.kernel,
    out_type=(
        jax.ShapeDtypeStruct((NPAD,), jnp.float32),
        jax.ShapeDtypeStruct((NPAD,), jnp.float32),
    ),
    mesh=_MESH,
    scratch_types=[
        pltpu.VMEM((2, CHP), jnp.int32),
        pltpu.VMEM((HALF,), jnp.float32),
        pltpu.VMEM((NS * NT,), jnp.float32),
        pltpu.VMEM((NT, D), jnp.float32),
        pltpu.VMEM((NPAD,), jnp.float32),
        pltpu.VMEM((D,), jnp.float32),
        pltpu.VMEM((NT,), jnp.float32),
        pltpu.VMEM((NT,), jnp.float32),
        pltpu.VMEM((NT,), jnp.float32),
        pltpu.VMEM((NT,), jnp.float32),
        pltpu.VMEM((L,), jnp.float32),
        pltpu.VMEM_SHARED((NS * HALF,), jnp.float32),
        pltpu.SemaphoreType.DMA,
        pltpu.SemaphoreType.DMA,
        pltpu.SemaphoreType.DMA,
        pltpu.SemaphoreType.REGULAR,
    ],
    compiler_params=pltpu.CompilerParams(needs_layout_passes=False,
                                         use_tc_tiling_on_sc=True),
)
def _sc_gcn(x_hbm, w_hbm, ei_hbm, b_hbm, out_hbm, g_hbm,
            ev2, hist, red, xsl, gv, wsl, hsl, gsl, dsl, osl, bv,
            shared, semx, seme, semc, semg):
    c = lax.axis_index("c")
    s = lax.axis_index("s")
    base = c * HALF
    row0 = base + s * NT
    a0 = (s * ECHUNK) // 128 * 128
    off = s * ECHUNK - a0

    full = row0 + NT <= N
    part = jnp.logical_not(full) & (row0 < N)

    @pl.when(full)
    def _():
        pltpu.async_copy(x_hbm.at[pl.ds(row0, NT), :], xsl, semx)

    @pl.when(part)
    def _():
        pltpu.async_copy(x_hbm.at[pl.ds(row0, _PART_ROWS), :],
                         xsl.at[pl.ds(0, _PART_ROWS), :], semx)

    pltpu.async_copy(ei_hbm.at[:, pl.ds(a0, CHP)], ev2, seme)
    pltpu.sync_copy(w_hbm, wsl)
    pltpu.sync_copy(b_hbm, bv)

    zero16 = jnp.zeros((L,), jnp.float32)

    @plsc.parallel_loop(0, HALF // L, unroll=8)
    def _(i):
        hist[pl.ds(i * L, L)] = zero16

    pltpu.make_async_copy(ei_hbm.at[:, pl.ds(a0, CHP)], ev2, seme).wait()
    ones = jnp.ones((L,), jnp.float32)

    @plsc.parallel_loop(0, EV, unroll=4)
    def _(i):
        cols = ev2[1, pl.ds(off + i * L, L)]
        loc = cols - base
        m = lax.bitcast_convert_type(loc, jnp.uint32) < jnp.uint32(HALF)
        plsc.addupdate_scatter(hist, [loc], ones, mask=m)

    pltpu.async_copy(hist, shared.at[pl.ds(s * HALF, HALF)], semc)

    @pl.when(full)
    def _():
        pltpu.make_async_copy(x_hbm.at[pl.ds(row0, NT), :], xsl, semx).wait()

    @pl.when(part)
    def _():
        pltpu.make_async_copy(x_hbm.at[pl.ds(row0, _PART_ROWS), :],
                              xsl.at[pl.ds(0, _PART_ROWS), :], semx).wait()

    lanes = lax.iota(jnp.int32, L)
    wvecs = [wsl[pl.ds(k * L, L)] for k in range(D // L)]

    @plsc.parallel_loop(0, NT, unroll=2)
    def _(r):
        a0 = xsl[r, pl.ds(0, L)] * wvecs[0]
        a1 = xsl[r, pl.ds(L, L)] * wvecs[1]
        for k in range(2, D // L, 2):
            a0 = a0 + xsl[r, pl.ds(k * L, L)] * wvecs[k]
            a1 = a1 + xsl[r, pl.ds((k + 1) * L, L)] * wvecs[k + 1]
        red[pl.ds(r * L, L)] = plsc.cumsum(a0 + a1)

    @plsc.parallel_loop(0, NTV)
    def _(j):
        idx = (lanes + j * L) * L + (L - 1)
        hsl[pl.ds(j * L, L)] = plsc.load_gather(red, [idx])

    pltpu.make_async_copy(hist, shared.at[pl.ds(s * HALF, HALF)], semc).wait()
    plsc.subcore_barrier()
    for t in range(NS):
        pltpu.async_copy(shared.at[pl.ds(t * HALF + s * NT, NT)],
                         red.at[pl.ds(t * NT, NT)], semc)
    for t in range(NS):
        pltpu.make_async_copy(shared.at[pl.ds(t * HALF + s * NT, NT)],
                              red.at[pl.ds(t * NT, NT)], semc).wait()

    @plsc.parallel_loop(0, NTV)
    def _(j):
        acc = red[pl.ds(j * L, L)]
        for t in range(1, NS):
            acc = acc + red[pl.ds(t * NT + j * L, L)]
        d = acc + 1.0
        y = _rsqrt_f32(d)
        dsl[pl.ds(j * L, L)] = y
        gsl[pl.ds(j * L, L)] = y * hsl[pl.ds(j * L, L)]

    pltpu.sync_copy(gsl, g_hbm.at[pl.ds(row0, NT)])

    plsc.subcore_barrier()

    @pl.when(s == 0)
    def _():
        pltpu.semaphore_signal(semg, 1, core_index=1 - c)
        pl.semaphore_wait(semg, 1)

    plsc.subcore_barrier()

    pltpu.async_copy(g_hbm, gv, semx)

    @plsc.parallel_loop(0, HALF // L, unroll=8)
    def _(i):
        hist[pl.ds(i * L, L)] = zero16

    pltpu.make_async_copy(g_hbm, gv, semx).wait()

    @plsc.parallel_loop(0, EV, unroll=4)
    def _(i):
        rows = ev2[0, pl.ds(off + i * L, L)]
        cols = ev2[1, pl.ds(off + i * L, L)]
        gvals = plsc.load_gather(gv, [rows])
        loc = cols - base
        m = lax.bitcast_convert_type(loc, jnp.uint32) < jnp.uint32(HALF)
        plsc.addupdate_scatter(hist, [loc], gvals, mask=m)

    pltpu.sync_copy(hist, shared.at[pl.ds(s * HALF, HALF)])
    plsc.subcore_barrier()
    for t in range(NS):
        pltpu.async_copy(shared.at[pl.ds(t * HALF + s * NT, NT)],
                         red.at[pl.ds(t * NT, NT)], semc)
    for t in range(NS):
        pltpu.make_async_copy(shared.at[pl.ds(t * HALF + s * NT, NT)],
                              red.at[pl.ds(t * NT, NT)], semc).wait()

    bval = bv[pl.ds(0, L)]

    @plsc.parallel_loop(0, NTV)
    def _(j):
        acc = red[pl.ds(j * L, L)]
        for t in range(1, NS):
            acc = acc + red[pl.ds(t * NT + j * L, L)]
        y = dsl[pl.ds(j * L, L)]
        o = y * acc + y * gsl[pl.ds(j * L, L)] + bval
        osl[pl.ds(j * L, L)] = jnp.maximum(o, 0.0)

    pltpu.sync_copy(osl, out_hbm.at[pl.ds(row0, NT)])


@jax.jit
def kernel(x, edge_index, W, b):
    ei = edge_index.astype(jnp.int32)
    w1 = W.astype(jnp.float32).reshape(D)
    b16 = jnp.broadcast_to(b.astype(jnp.float32).reshape(1), (L,))

    out_pad, _ = _sc_gcn(x, w1, ei, b16)
    return out_pad[:N].reshape(N, 1)
